# baseline (speedup 1.0000x reference)
import functools

import jax
import jax.numpy as jnp
from jax import lax
from jax.experimental import pallas as pl
from jax.experimental.pallas import tpu as pltpu
from jax.experimental.pallas import tpu_sc as plsc

VOCAB = 100000
EMBED_DIM = 64
CONTEXT = 20
BATCH = 1024
HIDDEN = 128

NC = 2
NS = 16
NW = NC * NS
B_PER_W = BATCH // NW
ROWS_PER_W = B_PER_W * CONTEXT
IDX_CHUNK = 128
N_CHUNKS = ROWS_PER_W // IDX_CHUNK

GB = 4096
VB = 2048


def _gproj_kernel(embt_ref, w1_ref, g_ref):
    g_ref[...] = lax.dot_general(
        embt_ref[...], w1_ref[...],
        (((0,), (0,)), ((), ())),
        preferred_element_type=jnp.float32,
    )


def _gproj(embT, W1):
    return pl.pallas_call(
        _gproj_kernel,
        grid=(pl.cdiv(VOCAB, GB),),
        in_specs=[
            pl.BlockSpec((EMBED_DIM, GB), lambda j: (0, j)),
            pl.BlockSpec((EMBED_DIM, HIDDEN), lambda j: (0, 0)),
        ],
        out_specs=pl.BlockSpec((GB, HIDDEN), lambda j: (j, 0)),
        out_shape=jax.ShapeDtypeStruct((VOCAB, HIDDEN), jnp.float32),
    )(embT, W1)


@functools.partial(
    pl.kernel,
    mesh=plsc.VectorSubcoreMesh(core_axis_name="c", subcore_axis_name="s"),
    out_type=jax.ShapeDtypeStruct((BATCH, HIDDEN), jnp.float32),
    scratch_types=[
        pltpu.VMEM((N_CHUNKS, IDX_CHUNK), jnp.int32),
        pltpu.VMEM((ROWS_PER_W, HIDDEN), jnp.float32),
        pltpu.VMEM((B_PER_W, HIDDEN), jnp.float32),
        pltpu.SemaphoreType.DMA,
    ],
)
def _gather_sum(idx_hbm, g_hbm, out_hbm, idx_v, rows_v, acc_v, sem):
    wid = lax.axis_index("s") * NC + lax.axis_index("c")
    pltpu.sync_copy(idx_hbm.at[wid], idx_v)
    copies = [
        pltpu.async_copy(
            g_hbm.at[idx_v.at[k]],
            rows_v.at[pl.ds(k * IDX_CHUNK, IDX_CHUNK)],
            sem,
        )
        for k in range(N_CHUNKS)
    ]

    def body(r, carry):
        for c in range(HIDDEN // 16):
            acc = rows_v[r * CONTEXT, pl.ds(c * 16, 16)]
            for j in range(1, CONTEXT):
                acc = acc + rows_v[r * CONTEXT + j, pl.ds(c * 16, 16)]
            acc_v[r, pl.ds(c * 16, 16)] = acc
        return carry

    for cp in copies:
        cp.wait()
    lax.fori_loop(0, B_PER_W, body, 0)
    pltpu.sync_copy(acc_v, out_hbm.at[pl.ds(wid * B_PER_W, B_PER_W)])


NBUF = 3
NSPLIT = 4
RSPLIT = VB // NSPLIT
NFULL = VOCAB // VB
TAIL = VOCAB - NFULL * VB
TSPLIT = TAIL // NSPLIT
GRID_B = NFULL + 1


def _mlp_kernel(
    hg_ref, b1_ref, w2t_ref, b2_ref, out_ref, hidt_ref, obuf_ref, sems, tsems
):
    j = pl.program_id(0)
    slot = lax.rem(j, NBUF)
    ones = jnp.ones((1, BATCH), jnp.float32)

    @pl.when(j == 0)
    def _():
        hid = jnp.maximum(hg_ref[...] + b1_ref[...], 0.0)
        hidt_ref[...] = hid.T

    @pl.when(j >= NBUF)
    def _():
        for k in range(NSPLIT):
            pltpu.make_async_copy(
                obuf_ref.at[slot, pl.ds(k * RSPLIT, RSPLIT), :],
                out_ref.at[pl.ds(k * RSPLIT, RSPLIT), :],
                sems.at[slot, k],
            ).wait()

    res = lax.dot_general(
        w2t_ref[...], hidt_ref[...],
        (((1,), (0,)), ((), ())),
        preferred_element_type=jnp.float32,
    )
    b2col = lax.dot_general(
        b2_ref[...], ones,
        (((0,), (0,)), ((), ())),
        preferred_element_type=jnp.float32,
    )
    obuf_ref[slot] = res + b2col

    row0 = pl.multiple_of(j * VB, VB)

    @pl.when(j < NFULL)
    def _():
        for k in range(NSPLIT):
            pltpu.make_async_copy(
                obuf_ref.at[slot, pl.ds(k * RSPLIT, RSPLIT), :],
                out_ref.at[pl.ds(row0 + k * RSPLIT, RSPLIT), :],
                sems.at[slot, k],
            ).start()

    @pl.when(j == NFULL)
    def _():
        for k in range(NSPLIT):
            pltpu.make_async_copy(
                obuf_ref.at[slot, pl.ds(k * TSPLIT, TSPLIT), :],
                out_ref.at[pl.ds(NFULL * VB + k * TSPLIT, TSPLIT), :],
                tsems.at[k],
            ).start()
        for step in range(GRID_B - NBUF, GRID_B - 1):
            s = step % NBUF
            for k in range(NSPLIT):
                pltpu.make_async_copy(
                    obuf_ref.at[s, pl.ds(k * RSPLIT, RSPLIT), :],
                    out_ref.at[pl.ds(k * RSPLIT, RSPLIT), :],
                    sems.at[s, k],
                ).wait()
        for k in range(NSPLIT):
            pltpu.make_async_copy(
                obuf_ref.at[slot, pl.ds(k * TSPLIT, TSPLIT), :],
                out_ref.at[pl.ds(k * TSPLIT, TSPLIT), :],
                tsems.at[k],
            ).wait()


def _mlp(hidG, b1, W2t, b2):
    out_t = pl.pallas_call(
        _mlp_kernel,
        grid=(GRID_B,),
        in_specs=[
            pl.BlockSpec((BATCH, HIDDEN), lambda j: (0, 0)),
            pl.BlockSpec((1, HIDDEN), lambda j: (0, 0)),
            pl.BlockSpec((VB, HIDDEN), lambda j: (j, 0)),
            pl.BlockSpec((1, VB), lambda j: (0, j)),
        ],
        out_specs=pl.BlockSpec(memory_space=pltpu.MemorySpace.HBM),
        out_shape=jax.ShapeDtypeStruct((VOCAB, BATCH), jnp.float32),
        scratch_shapes=[
            pltpu.VMEM((HIDDEN, BATCH), jnp.float32),
            pltpu.VMEM((NBUF, VB, BATCH), jnp.float32),
            pltpu.SemaphoreType.DMA((NBUF, NSPLIT)),
            pltpu.SemaphoreType.DMA((NSPLIT,)),
        ],
    )(hidG, b1.reshape(1, HIDDEN), W2t, b2.reshape(1, VOCAB))
    return out_t.T


def kernel(inputs, emb, W1, b1, W2, b2):
    idx = inputs.astype(jnp.int32).reshape(NW, N_CHUNKS, IDX_CHUNK)
    G = _gproj(emb.T, W1)
    hidG = _gather_sum(idx, G)
    return _mlp(hidG, b1, W2.T, b2)


# speedup vs baseline: 1.0481x; 1.0481x over previous
import functools

import jax
import jax.numpy as jnp
from jax import lax
from jax.experimental import pallas as pl
from jax.experimental.pallas import tpu as pltpu
from jax.experimental.pallas import tpu_sc as plsc

VOCAB = 100000
EMBED_DIM = 64
CONTEXT = 20
BATCH = 1024
HIDDEN = 128

NC = 2
NS = 16
NW = NC * NS
B_PER_W = BATCH // NW
ROWS_PER_W = B_PER_W * CONTEXT
IDX_CHUNK = 128
N_CHUNKS = ROWS_PER_W // IDX_CHUNK

GB = 16384
VB = 2048


def _gproj_kernel(embt_ref, w1_ref, g_ref):
    g_ref[...] = lax.dot_general(
        embt_ref[...], w1_ref[...],
        (((0,), (0,)), ((), ())),
        preferred_element_type=jnp.float32,
    )


def _gproj(embT, W1):
    return pl.pallas_call(
        _gproj_kernel,
        grid=(pl.cdiv(VOCAB, GB),),
        in_specs=[
            pl.BlockSpec((EMBED_DIM, GB), lambda j: (0, j)),
            pl.BlockSpec((EMBED_DIM, HIDDEN), lambda j: (0, 0)),
        ],
        out_specs=pl.BlockSpec((GB, HIDDEN), lambda j: (j, 0)),
        out_shape=jax.ShapeDtypeStruct((VOCAB, HIDDEN), jnp.float32),
    )(embT, W1)


@functools.partial(
    pl.kernel,
    mesh=plsc.VectorSubcoreMesh(core_axis_name="c", subcore_axis_name="s"),
    out_type=jax.ShapeDtypeStruct((BATCH, HIDDEN), jnp.float32),
    scratch_types=[
        pltpu.VMEM((N_CHUNKS, IDX_CHUNK), jnp.int32),
        pltpu.VMEM((ROWS_PER_W, HIDDEN), jnp.float32),
        pltpu.VMEM((B_PER_W, HIDDEN), jnp.float32),
        pltpu.SemaphoreType.DMA,
    ],
)
def _gather_sum(idx_hbm, g_hbm, out_hbm, idx_v, rows_v, acc_v, sem):
    wid = lax.axis_index("s") * NC + lax.axis_index("c")
    pltpu.sync_copy(idx_hbm.at[wid], idx_v)
    copies = [
        pltpu.async_copy(
            g_hbm.at[idx_v.at[k]],
            rows_v.at[pl.ds(k * IDX_CHUNK, IDX_CHUNK)],
            sem,
        )
        for k in range(N_CHUNKS)
    ]

    def body(r, carry):
        for c in range(HIDDEN // 16):
            acc = rows_v[r * CONTEXT, pl.ds(c * 16, 16)]
            for j in range(1, CONTEXT):
                acc = acc + rows_v[r * CONTEXT + j, pl.ds(c * 16, 16)]
            acc_v[r, pl.ds(c * 16, 16)] = acc
        return carry

    for cp in copies:
        cp.wait()
    lax.fori_loop(0, B_PER_W, body, 0)
    pltpu.sync_copy(acc_v, out_hbm.at[pl.ds(wid * B_PER_W, B_PER_W)])


NBUF = 3
NSPLIT = 4
RSPLIT = VB // NSPLIT
NFULL = VOCAB // VB
TAIL = VOCAB - NFULL * VB
TSPLIT = TAIL // NSPLIT
GRID_B = NFULL + 1


def _mlp_kernel(
    hg_ref, b1_ref, w2t_ref, b2_ref, out_ref, hidt_ref, obuf_ref, sems, tsems
):
    j = pl.program_id(0)
    slot = lax.rem(j, NBUF)
    ones = jnp.ones((1, BATCH), jnp.float32)

    @pl.when(j == 0)
    def _():
        hid = jnp.maximum(hg_ref[...] + b1_ref[...], 0.0)
        hidt_ref[...] = hid.T

    @pl.when(j >= NBUF)
    def _():
        for k in range(NSPLIT):
            pltpu.make_async_copy(
                obuf_ref.at[slot, pl.ds(k * RSPLIT, RSPLIT), :],
                out_ref.at[pl.ds(k * RSPLIT, RSPLIT), :],
                sems.at[slot, k],
            ).wait()

    res = lax.dot_general(
        w2t_ref[...], hidt_ref[...],
        (((1,), (0,)), ((), ())),
        preferred_element_type=jnp.float32,
    )
    b2col = lax.dot_general(
        b2_ref[...], ones,
        (((0,), (0,)), ((), ())),
        preferred_element_type=jnp.float32,
    )
    obuf_ref[slot] = res + b2col

    row0 = pl.multiple_of(j * VB, VB)

    @pl.when(j < NFULL)
    def _():
        for k in range(NSPLIT):
            pltpu.make_async_copy(
                obuf_ref.at[slot, pl.ds(k * RSPLIT, RSPLIT), :],
                out_ref.at[pl.ds(row0 + k * RSPLIT, RSPLIT), :],
                sems.at[slot, k],
            ).start()

    @pl.when(j == NFULL)
    def _():
        for k in range(NSPLIT):
            pltpu.make_async_copy(
                obuf_ref.at[slot, pl.ds(k * TSPLIT, TSPLIT), :],
                out_ref.at[pl.ds(NFULL * VB + k * TSPLIT, TSPLIT), :],
                tsems.at[k],
            ).start()
        for step in range(GRID_B - NBUF, GRID_B - 1):
            s = step % NBUF
            for k in range(NSPLIT):
                pltpu.make_async_copy(
                    obuf_ref.at[s, pl.ds(k * RSPLIT, RSPLIT), :],
                    out_ref.at[pl.ds(k * RSPLIT, RSPLIT), :],
                    sems.at[s, k],
                ).wait()
        for k in range(NSPLIT):
            pltpu.make_async_copy(
                obuf_ref.at[slot, pl.ds(k * TSPLIT, TSPLIT), :],
                out_ref.at[pl.ds(k * TSPLIT, TSPLIT), :],
                tsems.at[k],
            ).wait()


def _mlp(hidG, b1, W2t, b2):
    out_t = pl.pallas_call(
        _mlp_kernel,
        grid=(GRID_B,),
        in_specs=[
            pl.BlockSpec((BATCH, HIDDEN), lambda j: (0, 0)),
            pl.BlockSpec((1, HIDDEN), lambda j: (0, 0)),
            pl.BlockSpec((VB, HIDDEN), lambda j: (j, 0)),
            pl.BlockSpec((1, VB), lambda j: (0, j)),
        ],
        out_specs=pl.BlockSpec(memory_space=pltpu.MemorySpace.HBM),
        out_shape=jax.ShapeDtypeStruct((VOCAB, BATCH), jnp.float32),
        scratch_shapes=[
            pltpu.VMEM((HIDDEN, BATCH), jnp.float32),
            pltpu.VMEM((NBUF, VB, BATCH), jnp.float32),
            pltpu.SemaphoreType.DMA((NBUF, NSPLIT)),
            pltpu.SemaphoreType.DMA((NSPLIT,)),
        ],
    )(hidG, b1.reshape(1, HIDDEN), W2t, b2.reshape(1, VOCAB))
    return out_t.T


def kernel(inputs, emb, W1, b1, W2, b2):
    idx = inputs.astype(jnp.int32).reshape(NW, N_CHUNKS, IDX_CHUNK)
    G = _gproj(emb.T, W1)
    hidG = _gather_sum(idx, G)
    return _mlp(hidG, b1, W2.T, b2)


# speedup vs baseline: 1.0499x; 1.0018x over previous
import functools

import jax
import jax.numpy as jnp
from jax import lax
from jax.experimental import pallas as pl
from jax.experimental.pallas import tpu as pltpu
from jax.experimental.pallas import tpu_sc as plsc

VOCAB = 100000
EMBED_DIM = 64
CONTEXT = 20
BATCH = 1024
HIDDEN = 128

NC = 2
NS = 16
NW = NC * NS
B_PER_W = BATCH // NW
ROWS_PER_W = B_PER_W * CONTEXT
IDX_CHUNK = 128
N_CHUNKS = ROWS_PER_W // IDX_CHUNK

GB = 32768
VB = 2048


def _gproj_kernel(embt_ref, w1_ref, g_ref):
    g_ref[...] = lax.dot_general(
        embt_ref[...], w1_ref[...],
        (((0,), (0,)), ((), ())),
        preferred_element_type=jnp.float32,
    )


def _gproj(embT, W1):
    return pl.pallas_call(
        _gproj_kernel,
        grid=(pl.cdiv(VOCAB, GB),),
        in_specs=[
            pl.BlockSpec((EMBED_DIM, GB), lambda j: (0, j)),
            pl.BlockSpec((EMBED_DIM, HIDDEN), lambda j: (0, 0)),
        ],
        out_specs=pl.BlockSpec((GB, HIDDEN), lambda j: (j, 0)),
        out_shape=jax.ShapeDtypeStruct((VOCAB, HIDDEN), jnp.float32),
    )(embT, W1)


@functools.partial(
    pl.kernel,
    mesh=plsc.VectorSubcoreMesh(core_axis_name="c", subcore_axis_name="s"),
    out_type=jax.ShapeDtypeStruct((BATCH, HIDDEN), jnp.float32),
    scratch_types=[
        pltpu.VMEM((N_CHUNKS, IDX_CHUNK), jnp.int32),
        pltpu.VMEM((ROWS_PER_W, HIDDEN), jnp.float32),
        pltpu.VMEM((B_PER_W, HIDDEN), jnp.float32),
        pltpu.SemaphoreType.DMA,
    ],
)
def _gather_sum(idx_hbm, g_hbm, out_hbm, idx_v, rows_v, acc_v, sem):
    wid = lax.axis_index("s") * NC + lax.axis_index("c")
    pltpu.sync_copy(idx_hbm.at[wid], idx_v)
    copies = [
        pltpu.async_copy(
            g_hbm.at[idx_v.at[k]],
            rows_v.at[pl.ds(k * IDX_CHUNK, IDX_CHUNK)],
            sem,
        )
        for k in range(N_CHUNKS)
    ]

    def body(r, carry):
        for c in range(HIDDEN // 16):
            acc = rows_v[r * CONTEXT, pl.ds(c * 16, 16)]
            for j in range(1, CONTEXT):
                acc = acc + rows_v[r * CONTEXT + j, pl.ds(c * 16, 16)]
            acc_v[r, pl.ds(c * 16, 16)] = acc
        return carry

    for cp in copies:
        cp.wait()
    lax.fori_loop(0, B_PER_W, body, 0)
    pltpu.sync_copy(acc_v, out_hbm.at[pl.ds(wid * B_PER_W, B_PER_W)])


NBUF = 3
NSPLIT = 4
RSPLIT = VB // NSPLIT
NFULL = VOCAB // VB
TAIL = VOCAB - NFULL * VB
TSPLIT = TAIL // NSPLIT
GRID_B = NFULL + 1


def _mlp_kernel(
    hg_ref, b1_ref, w2t_ref, b2_ref, out_ref, hidt_ref, obuf_ref, sems, tsems
):
    j = pl.program_id(0)
    slot = lax.rem(j, NBUF)
    ones = jnp.ones((1, BATCH), jnp.float32)

    @pl.when(j == 0)
    def _():
        hid = jnp.maximum(hg_ref[...] + b1_ref[...], 0.0)
        hidt_ref[...] = hid.T

    @pl.when(j >= NBUF)
    def _():
        for k in range(NSPLIT):
            pltpu.make_async_copy(
                obuf_ref.at[slot, pl.ds(k * RSPLIT, RSPLIT), :],
                out_ref.at[pl.ds(k * RSPLIT, RSPLIT), :],
                sems.at[slot, k],
            ).wait()

    res = lax.dot_general(
        w2t_ref[...], hidt_ref[...],
        (((1,), (0,)), ((), ())),
        preferred_element_type=jnp.float32,
    )
    b2col = lax.dot_general(
        b2_ref[...], ones,
        (((0,), (0,)), ((), ())),
        preferred_element_type=jnp.float32,
    )
    obuf_ref[slot] = res + b2col

    row0 = pl.multiple_of(j * VB, VB)

    @pl.when(j < NFULL)
    def _():
        for k in range(NSPLIT):
            pltpu.make_async_copy(
                obuf_ref.at[slot, pl.ds(k * RSPLIT, RSPLIT), :],
                out_ref.at[pl.ds(row0 + k * RSPLIT, RSPLIT), :],
                sems.at[slot, k],
            ).start()

    @pl.when(j == NFULL)
    def _():
        for k in range(NSPLIT):
            pltpu.make_async_copy(
                obuf_ref.at[slot, pl.ds(k * TSPLIT, TSPLIT), :],
                out_ref.at[pl.ds(NFULL * VB + k * TSPLIT, TSPLIT), :],
                tsems.at[k],
            ).start()
        for step in range(GRID_B - NBUF, GRID_B - 1):
            s = step % NBUF
            for k in range(NSPLIT):
                pltpu.make_async_copy(
                    obuf_ref.at[s, pl.ds(k * RSPLIT, RSPLIT), :],
                    out_ref.at[pl.ds(k * RSPLIT, RSPLIT), :],
                    sems.at[s, k],
                ).wait()
        for k in range(NSPLIT):
            pltpu.make_async_copy(
                obuf_ref.at[slot, pl.ds(k * TSPLIT, TSPLIT), :],
                out_ref.at[pl.ds(k * TSPLIT, TSPLIT), :],
                tsems.at[k],
            ).wait()


def _mlp(hidG, b1, W2t, b2):
    out_t = pl.pallas_call(
        _mlp_kernel,
        grid=(GRID_B,),
        in_specs=[
            pl.BlockSpec((BATCH, HIDDEN), lambda j: (0, 0)),
            pl.BlockSpec((1, HIDDEN), lambda j: (0, 0)),
            pl.BlockSpec((VB, HIDDEN), lambda j: (j, 0)),
            pl.BlockSpec((1, VB), lambda j: (0, j)),
        ],
        out_specs=pl.BlockSpec(memory_space=pltpu.MemorySpace.HBM),
        out_shape=jax.ShapeDtypeStruct((VOCAB, BATCH), jnp.float32),
        scratch_shapes=[
            pltpu.VMEM((HIDDEN, BATCH), jnp.float32),
            pltpu.VMEM((NBUF, VB, BATCH), jnp.float32),
            pltpu.SemaphoreType.DMA((NBUF, NSPLIT)),
            pltpu.SemaphoreType.DMA((NSPLIT,)),
        ],
    )(hidG, b1.reshape(1, HIDDEN), W2t, b2.reshape(1, VOCAB))
    return out_t.T


def kernel(inputs, emb, W1, b1, W2, b2):
    idx = inputs.astype(jnp.int32).reshape(NW, N_CHUNKS, IDX_CHUNK)
    G = _gproj(emb.T, W1)
    hidG = _gather_sum(idx, G)
    return _mlp(hidG, b1, W2.T, b2)


# speedup vs baseline: 1.0593x; 1.0089x over previous
import functools

import jax
import jax.numpy as jnp
from jax import lax
from jax.experimental import pallas as pl
from jax.experimental.pallas import tpu as pltpu
from jax.experimental.pallas import tpu_sc as plsc

VOCAB = 100000
EMBED_DIM = 64
CONTEXT = 20
BATCH = 1024
HIDDEN = 128

NC = 2
NS = 16
NW = NC * NS
B_PER_W = BATCH // NW
ROWS_PER_W = B_PER_W * CONTEXT
IDX_CHUNK = 128
N_CHUNKS = ROWS_PER_W // IDX_CHUNK

GB = 32768
VB = 4096


def _gproj_kernel(embt_ref, w1_ref, g_ref):
    g_ref[...] = lax.dot_general(
        embt_ref[...], w1_ref[...],
        (((0,), (0,)), ((), ())),
        preferred_element_type=jnp.float32,
    )


def _gproj(embT, W1):
    return pl.pallas_call(
        _gproj_kernel,
        grid=(pl.cdiv(VOCAB, GB),),
        in_specs=[
            pl.BlockSpec((EMBED_DIM, GB), lambda j: (0, j)),
            pl.BlockSpec((EMBED_DIM, HIDDEN), lambda j: (0, 0)),
        ],
        out_specs=pl.BlockSpec((GB, HIDDEN), lambda j: (j, 0)),
        out_shape=jax.ShapeDtypeStruct((VOCAB, HIDDEN), jnp.float32),
    )(embT, W1)


@functools.partial(
    pl.kernel,
    mesh=plsc.VectorSubcoreMesh(core_axis_name="c", subcore_axis_name="s"),
    out_type=jax.ShapeDtypeStruct((BATCH, HIDDEN), jnp.float32),
    scratch_types=[
        pltpu.VMEM((N_CHUNKS, IDX_CHUNK), jnp.int32),
        pltpu.VMEM((ROWS_PER_W, HIDDEN), jnp.float32),
        pltpu.VMEM((B_PER_W, HIDDEN), jnp.float32),
        pltpu.SemaphoreType.DMA,
    ],
)
def _gather_sum(idx_hbm, g_hbm, out_hbm, idx_v, rows_v, acc_v, sem):
    wid = lax.axis_index("s") * NC + lax.axis_index("c")
    pltpu.sync_copy(idx_hbm.at[wid], idx_v)
    copies = [
        pltpu.async_copy(
            g_hbm.at[idx_v.at[k]],
            rows_v.at[pl.ds(k * IDX_CHUNK, IDX_CHUNK)],
            sem,
        )
        for k in range(N_CHUNKS)
    ]

    def body(r, carry):
        for c in range(HIDDEN // 16):
            acc = rows_v[r * CONTEXT, pl.ds(c * 16, 16)]
            for j in range(1, CONTEXT):
                acc = acc + rows_v[r * CONTEXT + j, pl.ds(c * 16, 16)]
            acc_v[r, pl.ds(c * 16, 16)] = acc
        return carry

    for cp in copies:
        cp.wait()
    lax.fori_loop(0, B_PER_W, body, 0)
    pltpu.sync_copy(acc_v, out_hbm.at[pl.ds(wid * B_PER_W, B_PER_W)])


NBUF = 2
NSPLIT = 4
RSPLIT = VB // NSPLIT
NFULL = VOCAB // VB
TAIL = VOCAB - NFULL * VB
TSPLIT = TAIL // NSPLIT
GRID_B = NFULL + 1


def _mlp_kernel(
    hg_ref, b1_ref, w2t_ref, b2_ref, out_ref, hidt_ref, obuf_ref, sems, tsems
):
    j = pl.program_id(0)
    slot = lax.rem(j, NBUF)
    ones = jnp.ones((1, BATCH), jnp.float32)

    @pl.when(j == 0)
    def _():
        hid = jnp.maximum(hg_ref[...] + b1_ref[...], 0.0)
        hidt_ref[...] = hid.T

    @pl.when(j >= NBUF)
    def _():
        for k in range(NSPLIT):
            pltpu.make_async_copy(
                obuf_ref.at[slot, pl.ds(k * RSPLIT, RSPLIT), :],
                out_ref.at[pl.ds(k * RSPLIT, RSPLIT), :],
                sems.at[slot, k],
            ).wait()

    res = lax.dot_general(
        w2t_ref[...], hidt_ref[...],
        (((1,), (0,)), ((), ())),
        preferred_element_type=jnp.float32,
    )
    b2col = lax.dot_general(
        b2_ref[...], ones,
        (((0,), (0,)), ((), ())),
        preferred_element_type=jnp.float32,
    )
    obuf_ref[slot] = res + b2col

    row0 = pl.multiple_of(j * VB, VB)

    @pl.when(j < NFULL)
    def _():
        for k in range(NSPLIT):
            pltpu.make_async_copy(
                obuf_ref.at[slot, pl.ds(k * RSPLIT, RSPLIT), :],
                out_ref.at[pl.ds(row0 + k * RSPLIT, RSPLIT), :],
                sems.at[slot, k],
            ).start()

    @pl.when(j == NFULL)
    def _():
        for k in range(NSPLIT):
            pltpu.make_async_copy(
                obuf_ref.at[slot, pl.ds(k * TSPLIT, TSPLIT), :],
                out_ref.at[pl.ds(NFULL * VB + k * TSPLIT, TSPLIT), :],
                tsems.at[k],
            ).start()
        for step in range(GRID_B - NBUF, GRID_B - 1):
            s = step % NBUF
            for k in range(NSPLIT):
                pltpu.make_async_copy(
                    obuf_ref.at[s, pl.ds(k * RSPLIT, RSPLIT), :],
                    out_ref.at[pl.ds(k * RSPLIT, RSPLIT), :],
                    sems.at[s, k],
                ).wait()
        for k in range(NSPLIT):
            pltpu.make_async_copy(
                obuf_ref.at[slot, pl.ds(k * TSPLIT, TSPLIT), :],
                out_ref.at[pl.ds(k * TSPLIT, TSPLIT), :],
                tsems.at[k],
            ).wait()


def _mlp(hidG, b1, W2t, b2):
    out_t = pl.pallas_call(
        _mlp_kernel,
        grid=(GRID_B,),
        in_specs=[
            pl.BlockSpec((BATCH, HIDDEN), lambda j: (0, 0)),
            pl.BlockSpec((1, HIDDEN), lambda j: (0, 0)),
            pl.BlockSpec((VB, HIDDEN), lambda j: (j, 0)),
            pl.BlockSpec((1, VB), lambda j: (0, j)),
        ],
        out_specs=pl.BlockSpec(memory_space=pltpu.MemorySpace.HBM),
        out_shape=jax.ShapeDtypeStruct((VOCAB, BATCH), jnp.float32),
        scratch_shapes=[
            pltpu.VMEM((HIDDEN, BATCH), jnp.float32),
            pltpu.VMEM((NBUF, VB, BATCH), jnp.float32),
            pltpu.SemaphoreType.DMA((NBUF, NSPLIT)),
            pltpu.SemaphoreType.DMA((NSPLIT,)),
        ],
    )(hidG, b1.reshape(1, HIDDEN), W2t, b2.reshape(1, VOCAB))
    return out_t.T


def kernel(inputs, emb, W1, b1, W2, b2):
    idx = inputs.astype(jnp.int32).reshape(NW, N_CHUNKS, IDX_CHUNK)
    G = _gproj(emb.T, W1)
    hidG = _gather_sum(idx, G)
    return _mlp(hidG, b1, W2.T, b2)
